# Initial kernel scaffold; baseline (speedup 1.0000x reference)
import functools

import jax
import jax.numpy as jnp
from jax import lax
from jax.experimental import pallas as pl
from jax.experimental.pallas import tpu as pltpu
from jax.experimental.pallas import tpu_sc as plsc

N = 10000
E = 320000
D = 128

NC = 2
NS = 16
NW = NC * NS
EPW = E // NW
CHUNK = 80
NSTEPS = EPW // CHUNK
ROWS_PER_TILE = N // NS


def _bn_body(x_ref, g_ref, b_ref, o_ref):
    x = x_ref[...]
    mean = jnp.mean(x, axis=0, keepdims=True)
    var = jnp.mean((x - mean) ** 2, axis=0, keepdims=True)
    o_ref[...] = (x - mean) * lax.rsqrt(var + 1e-5) * g_ref[...] + b_ref[...]


def _batchnorm(X, gamma, beta):
    return pl.pallas_call(
        _bn_body,
        out_shape=jax.ShapeDtypeStruct((N, D), jnp.float32),
    )(X, gamma.reshape(1, D), beta.reshape(1, D))


def _edge_mm_body(a_ref, w_ref, b_ref, o_ref):
    o_ref[...] = jnp.dot(a_ref[...], w_ref[...],
                         preferred_element_type=jnp.float32) + b_ref[...]


_BLK_E = 2560


def _edge_mm(attr, w_t, b):
    return pl.pallas_call(
        _edge_mm_body,
        grid=(E // _BLK_E,),
        in_specs=[pl.BlockSpec((_BLK_E, D), lambda i: (i, 0)),
                  pl.BlockSpec((D, D), lambda i: (0, 0)),
                  pl.BlockSpec((1, D), lambda i: (0, 0))],
        out_specs=pl.BlockSpec((_BLK_E, D), lambda i: (i, 0)),
        out_shape=jax.ShapeDtypeStruct((E, D), jnp.float32),
    )(attr, w_t, b)


def _node_body(x_ref, agg_ref, w_ref, b_ref, o_ref):
    h = x_ref[...] + agg_ref[0] + agg_ref[1]
    o_ref[...] = jnp.tanh(
        jnp.dot(h, w_ref[...], preferred_element_type=jnp.float32) + b_ref[...])


def _node_update(x, agg, w_t, b):
    return pl.pallas_call(
        _node_body,
        out_shape=jax.ShapeDtypeStruct((N, D), jnp.float32),
    )(x, agg, w_t, b.reshape(1, D))


def _final_body(x1_ref, agg_ref, w2_ref, b2_ref, fc_ref, o_ref):
    x1 = x1_ref[...]
    h = x1 + agg_ref[0] + agg_ref[1]
    x2 = jnp.tanh(
        jnp.dot(h, w2_ref[...], preferred_element_type=jnp.float32) + b2_ref[...])
    x3 = jnp.tanh(jnp.dot(x2, fc_ref[...], preferred_element_type=jnp.float32))
    o_ref[...] = jnp.concatenate([x1, x2, x3], axis=-1)


def _final(x1, agg, w2_t, b2, fc_t):
    return pl.pallas_call(
        _final_body,
        out_shape=jax.ShapeDtypeStruct((N, 3 * D), jnp.float32),
    )(x1, agg, w2_t, b2.reshape(1, D), fc_t)


def _sc_scatter_body(src_hbm, dst_hbm, ea_hbm, x_hbm, zero_hbm, out_hbm,
                     agg_sp, src_v, dst_v, xg_v, ea_v, sem):
    cid = lax.axis_index("c")
    sid = lax.axis_index("s")
    wid = cid * NS + sid
    base_row = sid * ROWS_PER_TILE

    pltpu.sync_copy(zero_hbm.at[pl.ds(base_row, ROWS_PER_TILE)],
                    agg_sp.at[pl.ds(base_row, ROWS_PER_TILE)])
    plsc.subcore_barrier()

    ebase = wid * EPW

    @pl.loop(0, NSTEPS)
    def _step(step):
        off = ebase + step * CHUNK
        pltpu.sync_copy(src_hbm.at[pl.ds(off, CHUNK)], src_v)
        pltpu.sync_copy(dst_hbm.at[pl.ds(off, CHUNK)], dst_v)
        pltpu.async_copy(x_hbm.at[src_v], xg_v, sem).wait()
        pltpu.sync_copy(ea_hbm.at[pl.ds(off, CHUNK)], ea_v)

        @pl.loop(0, CHUNK)
        def _row(r):
            @pl.loop(0, D, step=16)
            def _col(c):
                slc = (pl.ds(r, 1), pl.ds(c, 16))
                ea_v[slc] = jnp.maximum(ea_v[slc] + xg_v[slc], 0.0)

        pltpu.sync_copy(ea_v, agg_sp.at[dst_v], add=True)

    plsc.subcore_barrier()
    pltpu.sync_copy(agg_sp.at[pl.ds(base_row, ROWS_PER_TILE)],
                    out_hbm.at[cid, pl.ds(base_row, ROWS_PER_TILE)])


def _sc_scatter(src, dst, ea, x, zeros):
    mesh = plsc.VectorSubcoreMesh(core_axis_name="c", subcore_axis_name="s")
    run = functools.partial(
        pl.kernel,
        out_type=jax.ShapeDtypeStruct((NC, N, D), jnp.float32),
        mesh=mesh,
        scratch_types=[
            pltpu.VMEM_SHARED((N, D), jnp.float32),
            pltpu.VMEM((CHUNK,), jnp.int32),
            pltpu.VMEM((CHUNK,), jnp.int32),
            pltpu.VMEM((CHUNK, D), jnp.float32),
            pltpu.VMEM((CHUNK, D), jnp.float32),
            pltpu.SemaphoreType.DMA,
        ],
    )(_sc_scatter_body)
    return run(src, dst, ea, x, zeros)


def kernel(X, edge_index, edge_attr, bn_gamma, bn_beta,
           lin1e_w, lin1e_b, nn1_w, nn1_b,
           lin2e_w, lin2e_b, nn2_w, nn2_b, fc1_w):
    src = edge_index[0].astype(jnp.int32)
    dst = edge_index[1].astype(jnp.int32)
    zeros = jnp.zeros((N, D), jnp.float32)

    x = _batchnorm(X, bn_gamma, bn_beta)
    ea1 = _edge_mm(edge_attr, lin1e_w.T, lin1e_b.reshape(1, D))
    agg1 = _sc_scatter(src, dst, ea1, x, zeros)
    ea2 = _edge_mm(edge_attr, lin2e_w.T, lin2e_b.reshape(1, D))
    x1 = _node_update(x, agg1, nn1_w.T, nn1_b)
    agg2 = _sc_scatter(src, dst, ea2, x1, zeros)
    return _final(x1, agg2, nn2_w.T, nn2_b, fc1_w.T)


# speedup vs baseline: 2.6673x; 2.6673x over previous
import functools

import jax
import jax.numpy as jnp
from jax import lax
from jax.experimental import pallas as pl
from jax.experimental.pallas import tpu as pltpu
from jax.experimental.pallas import tpu_sc as plsc

N = 10000
E = 320000
D = 128

NC = 2
NS = 16
NW = NC * NS
EPW = E // NW
CHUNK = 80
NSTEPS = EPW // CHUNK
NPAD = 10240
ROWS_PER_TILE = NPAD // NS


def _bn_body(x_ref, g_ref, b_ref, o_ref):
    x = x_ref[...]
    mean = jnp.mean(x, axis=0, keepdims=True)
    var = jnp.mean((x - mean) ** 2, axis=0, keepdims=True)
    o_ref[...] = (x - mean) * lax.rsqrt(var + 1e-5) * g_ref[...] + b_ref[...]


def _batchnorm(X, gamma, beta):
    return pl.pallas_call(
        _bn_body,
        out_shape=jax.ShapeDtypeStruct((N, D), jnp.float32),
    )(X, gamma.reshape(1, D), beta.reshape(1, D))


def _edge_mm_body(a_ref, w_ref, b_ref, o_ref):
    o_ref[...] = jnp.dot(a_ref[...], w_ref[...],
                         preferred_element_type=jnp.float32) + b_ref[...]


_BLK_E = 2560


def _edge_mm(attr, w_t, b):
    return pl.pallas_call(
        _edge_mm_body,
        grid=(E // _BLK_E,),
        in_specs=[pl.BlockSpec((_BLK_E, D), lambda i: (i, 0)),
                  pl.BlockSpec((D, D), lambda i: (0, 0)),
                  pl.BlockSpec((1, D), lambda i: (0, 0))],
        out_specs=pl.BlockSpec((_BLK_E, D), lambda i: (i, 0)),
        out_shape=jax.ShapeDtypeStruct((E, D), jnp.float32),
    )(attr, w_t, b)


def _node_body(x_ref, agg_ref, w_ref, b_ref, o_ref):
    h = x_ref[...] + agg_ref[0] + agg_ref[1]
    o_ref[...] = jnp.tanh(
        jnp.dot(h, w_ref[...], preferred_element_type=jnp.float32) + b_ref[...])


def _node_update(x, agg, w_t, b):
    return pl.pallas_call(
        _node_body,
        grid=(1,),
        in_specs=[pl.BlockSpec((N, D), lambda i: (0, 0)),
                  pl.BlockSpec((NC, N, D), lambda i: (0, 0, 0)),
                  pl.BlockSpec((D, D), lambda i: (0, 0)),
                  pl.BlockSpec((1, D), lambda i: (0, 0))],
        out_specs=pl.BlockSpec((N, D), lambda i: (0, 0)),
        out_shape=jax.ShapeDtypeStruct((N, D), jnp.float32),
    )(x, agg, w_t, b.reshape(1, D))


def _final_body(x1_ref, agg_ref, w2_ref, b2_ref, fc_ref, o_ref):
    x1 = x1_ref[...]
    h = x1 + agg_ref[0] + agg_ref[1]
    x2 = jnp.tanh(
        jnp.dot(h, w2_ref[...], preferred_element_type=jnp.float32) + b2_ref[...])
    x3 = jnp.tanh(jnp.dot(x2, fc_ref[...], preferred_element_type=jnp.float32))
    o_ref[...] = jnp.concatenate([x1, x2, x3], axis=-1)


def _final(x1, agg, w2_t, b2, fc_t):
    return pl.pallas_call(
        _final_body,
        grid=(1,),
        in_specs=[pl.BlockSpec((N, D), lambda i: (0, 0)),
                  pl.BlockSpec((NC, N, D), lambda i: (0, 0, 0)),
                  pl.BlockSpec((D, D), lambda i: (0, 0)),
                  pl.BlockSpec((1, D), lambda i: (0, 0)),
                  pl.BlockSpec((D, D), lambda i: (0, 0))],
        out_specs=pl.BlockSpec((N, 3 * D), lambda i: (0, 0)),
        out_shape=jax.ShapeDtypeStruct((N, 3 * D), jnp.float32),
    )(x1, agg, w2_t, b2.reshape(1, D), fc_t)


def _sc_scatter_body(src_hbm, dst_hbm, ea_hbm, x_hbm, zero_hbm, out_hbm,
                     agg_sp, src_v, dst_v, xg_v, ea_v, sem):
    cid = lax.axis_index("c")
    sid = lax.axis_index("s")
    wid = cid * NS + sid
    base_row = sid * ROWS_PER_TILE

    pltpu.sync_copy(zero_hbm.at[pl.ds(base_row, ROWS_PER_TILE)],
                    agg_sp.at[pl.ds(base_row, ROWS_PER_TILE)])
    plsc.subcore_barrier()

    ebase = wid * EPW

    @pl.loop(0, NSTEPS)
    def _step(step):
        off = ebase + step * CHUNK
        pltpu.sync_copy(src_hbm.at[pl.ds(off, CHUNK)], src_v)
        pltpu.sync_copy(dst_hbm.at[pl.ds(off, CHUNK)], dst_v)
        pltpu.async_copy(x_hbm.at[src_v], xg_v, sem).wait()
        pltpu.sync_copy(ea_hbm.at[pl.ds(off, CHUNK)], ea_v)

        @pl.loop(0, CHUNK)
        def _row(r):
            @pl.loop(0, D, step=16)
            def _col(c):
                slc = (pl.ds(r, 1), pl.ds(c, 16))
                ea_v[slc] = jnp.maximum(ea_v[slc] + xg_v[slc], 0.0)

        pltpu.sync_copy(ea_v, agg_sp.at[dst_v], add=True)

    plsc.subcore_barrier()
    pltpu.sync_copy(agg_sp.at[pl.ds(base_row, ROWS_PER_TILE)],
                    out_hbm.at[cid, pl.ds(base_row, ROWS_PER_TILE)])


def _sc_scatter(src, dst, ea, x, zeros):
    mesh = plsc.VectorSubcoreMesh(core_axis_name="c", subcore_axis_name="s")
    run = functools.partial(
        pl.kernel,
        out_type=jax.ShapeDtypeStruct((NC, NPAD, D), jnp.float32),
        mesh=mesh,
        scratch_types=[
            pltpu.VMEM_SHARED((NPAD, D), jnp.float32),
            pltpu.VMEM((CHUNK,), jnp.int32),
            pltpu.VMEM((CHUNK,), jnp.int32),
            pltpu.VMEM((CHUNK, D), jnp.float32),
            pltpu.VMEM((CHUNK, D), jnp.float32),
            pltpu.SemaphoreType.DMA,
        ],
    )(_sc_scatter_body)
    return run(src, dst, ea, x, zeros)


def kernel(X, edge_index, edge_attr, bn_gamma, bn_beta,
           lin1e_w, lin1e_b, nn1_w, nn1_b,
           lin2e_w, lin2e_b, nn2_w, nn2_b, fc1_w):
    src = edge_index[0].astype(jnp.int32)
    dst = edge_index[1].astype(jnp.int32)
    zeros = jnp.zeros((NPAD, D), jnp.float32)

    x = _batchnorm(X, bn_gamma, bn_beta)
    ea1 = _edge_mm(edge_attr, lin1e_w.T, lin1e_b.reshape(1, D))
    agg1 = _sc_scatter(src, dst, ea1, x, zeros)
    ea2 = _edge_mm(edge_attr, lin2e_w.T, lin2e_b.reshape(1, D))
    x1 = _node_update(x, agg1, nn1_w.T, nn1_b)
    agg2 = _sc_scatter(src, dst, ea2, x1, zeros)
    return _final(x1, agg2, nn2_w.T, nn2_b, fc1_w.T)


# speedup vs baseline: 5.4036x; 2.0258x over previous
import functools

import jax
import jax.numpy as jnp
from jax import lax
from jax.experimental import pallas as pl
from jax.experimental.pallas import tpu as pltpu
from jax.experimental.pallas import tpu_sc as plsc

N = 10000
E = 320000
D = 128

NC = 2
NS = 16
NW = NC * NS
EPW = E // NW
CHUNK = 80
NSTEPS = EPW // CHUNK
NPAD = 10240
ROWS_PER_TILE = NPAD // NS


def _bn_body(x_ref, g_ref, b_ref, o_ref):
    x = x_ref[...]
    mean = jnp.mean(x, axis=0, keepdims=True)
    var = jnp.mean((x - mean) ** 2, axis=0, keepdims=True)
    o_ref[...] = (x - mean) * lax.rsqrt(var + 1e-5) * g_ref[...] + b_ref[...]


def _batchnorm(X, gamma, beta):
    return pl.pallas_call(
        _bn_body,
        out_shape=jax.ShapeDtypeStruct((N, D), jnp.float32),
    )(X, gamma.reshape(1, D), beta.reshape(1, D))


def _edge_mm_body(a_ref, w_ref, b_ref, o_ref):
    o_ref[...] = jnp.dot(a_ref[...], w_ref[...],
                         preferred_element_type=jnp.float32) + b_ref[...]


_BLK_E = 2560


def _edge_mm(attr, w_t, b):
    return pl.pallas_call(
        _edge_mm_body,
        grid=(E // _BLK_E,),
        in_specs=[pl.BlockSpec((_BLK_E, D), lambda i: (i, 0)),
                  pl.BlockSpec((D, D), lambda i: (0, 0)),
                  pl.BlockSpec((1, D), lambda i: (0, 0))],
        out_specs=pl.BlockSpec((_BLK_E, D), lambda i: (i, 0)),
        out_shape=jax.ShapeDtypeStruct((E, D), jnp.float32),
    )(attr, w_t, b)


def _node_body(x_ref, agg_ref, w_ref, b_ref, o_ref):
    h = x_ref[...] + agg_ref[0] + agg_ref[1]
    o_ref[...] = jnp.tanh(
        jnp.dot(h, w_ref[...], preferred_element_type=jnp.float32) + b_ref[...])


def _node_update(x, agg, w_t, b):
    return pl.pallas_call(
        _node_body,
        grid=(1,),
        in_specs=[pl.BlockSpec((N, D), lambda i: (0, 0)),
                  pl.BlockSpec((NC, N, D), lambda i: (0, 0, 0)),
                  pl.BlockSpec((D, D), lambda i: (0, 0)),
                  pl.BlockSpec((1, D), lambda i: (0, 0))],
        out_specs=pl.BlockSpec((N, D), lambda i: (0, 0)),
        out_shape=jax.ShapeDtypeStruct((N, D), jnp.float32),
    )(x, agg, w_t, b.reshape(1, D))


def _final_body(x1_ref, agg_ref, w2_ref, b2_ref, fc_ref, o_ref):
    x1 = x1_ref[...]
    h = x1 + agg_ref[0] + agg_ref[1]
    x2 = jnp.tanh(
        jnp.dot(h, w2_ref[...], preferred_element_type=jnp.float32) + b2_ref[...])
    x3 = jnp.tanh(jnp.dot(x2, fc_ref[...], preferred_element_type=jnp.float32))
    o_ref[...] = jnp.concatenate([x1, x2, x3], axis=-1)


def _final(x1, agg, w2_t, b2, fc_t):
    return pl.pallas_call(
        _final_body,
        grid=(1,),
        in_specs=[pl.BlockSpec((N, D), lambda i: (0, 0)),
                  pl.BlockSpec((NC, N, D), lambda i: (0, 0, 0)),
                  pl.BlockSpec((D, D), lambda i: (0, 0)),
                  pl.BlockSpec((1, D), lambda i: (0, 0)),
                  pl.BlockSpec((D, D), lambda i: (0, 0))],
        out_specs=pl.BlockSpec((N, 3 * D), lambda i: (0, 0)),
        out_shape=jax.ShapeDtypeStruct((N, 3 * D), jnp.float32),
    )(x1, agg, w2_t, b2.reshape(1, D), fc_t)


NDATA = 2
NIDX = 4


def _sc_scatter_body(src_hbm, dst_hbm, ea_hbm, x_hbm, zero_hbm, out_hbm,
                     agg_sp, src_ring, dst_ring, xg_bufs, ea_bufs,
                     gsems, esems, issems, idsems, zsem):
    cid = lax.axis_index("c")
    sid = lax.axis_index("s")
    wid = cid * NS + sid
    base_row = sid * ROWS_PER_TILE
    ebase = wid * EPW

    def _issue_idx(s, j):
        pltpu.async_copy(src_hbm.at[wid, s], src_ring.at[j], issems.at[j])
        pltpu.async_copy(dst_hbm.at[wid, s], dst_ring.at[j], idsems.at[j])

    def _wait_idx(s, j):
        pltpu.make_async_copy(src_hbm.at[wid, 0], src_ring.at[j],
                              issems.at[j]).wait()
        pltpu.make_async_copy(dst_hbm.at[wid, 0], dst_ring.at[j],
                              idsems.at[j]).wait()

    def _issue_data(s, b, j):
        pltpu.async_copy(x_hbm.at[src_ring.at[j]], xg_bufs.at[b], gsems.at[b])
        pltpu.async_copy(ea_hbm.at[pl.ds(ebase + s * CHUNK, CHUNK)],
                         ea_bufs.at[b], esems.at[b])

    def _wait_data(b):
        pltpu.make_async_copy(ea_hbm.at[pl.ds(0, CHUNK)], xg_bufs.at[b],
                              gsems.at[b]).wait()
        pltpu.make_async_copy(ea_hbm.at[pl.ds(0, CHUNK)], ea_bufs.at[b],
                              esems.at[b]).wait()

    pltpu.async_copy(zero_hbm.at[pl.ds(base_row, ROWS_PER_TILE)],
                     agg_sp.at[pl.ds(base_row, ROWS_PER_TILE)], zsem)
    for j in range(NIDX):
        _issue_idx(j, j)
    for b in range(NDATA):
        _wait_idx(b, b)
        _issue_data(b, b, b)
    pltpu.make_async_copy(zero_hbm.at[pl.ds(base_row, ROWS_PER_TILE)],
                          agg_sp.at[pl.ds(base_row, ROWS_PER_TILE)],
                          zsem).wait()
    plsc.subcore_barrier()

    @pl.loop(0, NSTEPS, step=NIDX)
    def _round(g):
        for b in range(NIDX):
            s = g + b
            db = b % NDATA

            @pl.when(s < NSTEPS)
            def _body():
                xg_b = xg_bufs.at[db]
                ea_b = ea_bufs.at[db]
                _wait_data(db)

                @pl.loop(0, CHUNK)
                def _row(r):
                    for c in range(0, D, 16):
                        slc = (pl.ds(r, 1), pl.ds(c, 16))
                        ea_b[slc] = jnp.maximum(ea_b[slc] + xg_b[slc], 0.0)

                pltpu.sync_copy(ea_b, agg_sp.at[dst_ring.at[b]], add=True)

                @pl.when(s + NIDX < NSTEPS)
                def _refill_idx():
                    _issue_idx(s + NIDX, b)

                @pl.when(s + NDATA < NSTEPS)
                def _refill_data():
                    j2 = (b + NDATA) % NIDX
                    _wait_idx(s + NDATA, j2)
                    _issue_data(s + NDATA, db, j2)

    plsc.subcore_barrier()
    pltpu.sync_copy(agg_sp.at[pl.ds(base_row, ROWS_PER_TILE)],
                    out_hbm.at[cid, pl.ds(base_row, ROWS_PER_TILE)])


def _sc_scatter(src, dst, ea, x, zeros):
    mesh = plsc.VectorSubcoreMesh(core_axis_name="c", subcore_axis_name="s")
    run = functools.partial(
        pl.kernel,
        out_type=jax.ShapeDtypeStruct((NC, NPAD, D), jnp.float32),
        mesh=mesh,
        scratch_types=[
            pltpu.VMEM_SHARED((NPAD, D), jnp.float32),
            pltpu.VMEM((NIDX, CHUNK), jnp.int32),
            pltpu.VMEM((NIDX, CHUNK), jnp.int32),
            pltpu.VMEM((NDATA, CHUNK, D), jnp.float32),
            pltpu.VMEM((NDATA, CHUNK, D), jnp.float32),
            pltpu.SemaphoreType.DMA((NDATA,)),
            pltpu.SemaphoreType.DMA((NDATA,)),
            pltpu.SemaphoreType.DMA((NIDX,)),
            pltpu.SemaphoreType.DMA((NIDX,)),
            pltpu.SemaphoreType.DMA,
        ],
    )(_sc_scatter_body)
    return run(src.reshape(NW, NSTEPS, CHUNK), dst.reshape(NW, NSTEPS, CHUNK),
               ea, x, zeros)


def kernel(X, edge_index, edge_attr, bn_gamma, bn_beta,
           lin1e_w, lin1e_b, nn1_w, nn1_b,
           lin2e_w, lin2e_b, nn2_w, nn2_b, fc1_w):
    src = edge_index[0].astype(jnp.int32)
    dst = edge_index[1].astype(jnp.int32)
    zeros = jnp.zeros((NPAD, D), jnp.float32)

    x = _batchnorm(X, bn_gamma, bn_beta)
    ea1 = _edge_mm(edge_attr, lin1e_w.T, lin1e_b.reshape(1, D))
    agg1 = _sc_scatter(src, dst, ea1, x, zeros)
    ea2 = _edge_mm(edge_attr, lin2e_w.T, lin2e_b.reshape(1, D))
    x1 = _node_update(x, agg1, nn1_w.T, nn1_b)
    agg2 = _sc_scatter(src, dst, ea2, x1, zeros)
    return _final(x1, agg2, nn2_w.T, nn2_b, fc1_w.T)
